# Initial kernel scaffold; baseline (speedup 1.0000x reference)
import jax
import jax.numpy as jnp
from jax.experimental import pallas as pl


def kernel(nodes_embedding, edges_embedding, edge_index, w_node, b_node, w_edge, b_edge):
    raise NotImplementedError("write your pallas kernel here")


# speedup vs baseline: 3.7034x; 3.7034x over previous
import math

import jax
import jax.numpy as jnp
from jax import lax
from jax.experimental import pallas as pl
from jax.experimental.pallas import tpu as pltpu

_LOG2E = 1.4426950408889634
_TEMPERATURE = 0.7


def _round_up(x, m):
    return ((x + m - 1) // m) * m


def _node_project_kernel(x_ref, w_ref, a_ref, b_ref):
    ab = jnp.dot(x_ref[...], w_ref[...], preferred_element_type=jnp.float32)
    d = a_ref.shape[1]
    a_ref[...] = ab[:, :d].astype(a_ref.dtype)
    b_ref[...] = ab[:, d:].astype(b_ref.dtype)


def _normalize_kernel(inv_t, m_actual, tb, mask_rows):
    s2 = math.sqrt(_LOG2E * inv_t)

    def _body(g0_ref, g1_ref, ee_ref, we_ref, bn_ref, be_ref,
              nhat_ref, ehat_ref, absin_ref):
        nodes_map = (g0_ref[...].astype(jnp.float32)
                     + g1_ref[...].astype(jnp.float32) + bn_ref[...])
        edges_map = jnp.dot(ee_ref[...], we_ref[...],
                            preferred_element_type=jnp.float32) + be_ref[...]

        n_sq = jnp.sum(nodes_map * nodes_map, axis=-1, keepdims=True)
        e_sq = jnp.sum(edges_map * edges_map, axis=-1, keepdims=True)
        n_scale = jnp.where(n_sq > 0.0, lax.rsqrt(n_sq), 0.0)
        e_scale = jnp.where(e_sq > 0.0, lax.rsqrt(e_sq), 0.0)

        if mask_rows:
            row0 = pl.program_id(0) * tb
            valid = (row0 + lax.broadcasted_iota(jnp.int32, (tb, 1), 0)
                     < m_actual).astype(jnp.float32)
            n_scale = n_scale * valid
            e_scale = e_scale * valid

        rowdot = jnp.sum(nodes_map * edges_map, axis=-1, keepdims=True)
        absin_ref[...] = jnp.abs(rowdot) * (n_scale * e_scale * inv_t)
        nhat_ref[...] = (nodes_map * (n_scale * s2)).astype(nhat_ref.dtype)
        ehat_ref[...] = (edges_map * (e_scale * s2)).astype(ehat_ref.dtype)

    return _body


def _similarity_kernel(tn):
    def _body(nhat_ref, ehat_ref, rowsum_ref, colpart_ref):
        j = pl.program_id(1)

        @pl.when(j == 0)
        def _():
            rowsum_ref[...] = jnp.zeros_like(rowsum_ref)

        off = pl.multiple_of(j * tn, tn)
        e_chunk = ehat_ref[pl.ds(off, tn), :]
        mat = lax.dot_general(
            nhat_ref[...], e_chunk,
            dimension_numbers=(((1,), (1,)), ((), ())),
            preferred_element_type=jnp.float32)
        mi = jnp.exp2(-jnp.abs(mat))

        rowsum_ref[...] += jnp.sum(mi, axis=1, keepdims=True)
        colpart = jnp.sum(mi, axis=0, keepdims=True)
        colpart_ref[...] = colpart.reshape(colpart_ref.shape)

    return _body


def kernel(nodes_embedding, edges_embedding, edge_index,
           w_node, b_node, w_edge, b_edge):
    n_nodes, e_dim = nodes_embedding.shape
    m = edge_index.shape[1]
    d = w_node.shape[1]
    inv_t = float(1.0 / _TEMPERATURE)

    tb_n = 256
    n_pad = _round_up(n_nodes, tb_n)
    nodes_bf = jnp.pad(nodes_embedding, ((0, n_pad - n_nodes), (0, 0))
                       ).astype(jnp.bfloat16)
    w_cat = jnp.concatenate([w_node[:e_dim], w_node[e_dim:]],
                            axis=1).astype(jnp.bfloat16)
    proj_a, proj_b = pl.pallas_call(
        _node_project_kernel,
        out_shape=(
            jax.ShapeDtypeStruct((n_pad, d), jnp.bfloat16),
            jax.ShapeDtypeStruct((n_pad, d), jnp.bfloat16),
        ),
        grid=(n_pad // tb_n,),
        in_specs=[
            pl.BlockSpec((tb_n, e_dim), lambda i: (i, 0)),
            pl.BlockSpec((e_dim, 2 * d), lambda i: (0, 0)),
        ],
        out_specs=(
            pl.BlockSpec((tb_n, d), lambda i: (i, 0)),
            pl.BlockSpec((tb_n, d), lambda i: (i, 0)),
        ),
        compiler_params=pltpu.CompilerParams(
            dimension_semantics=("parallel",),
            vmem_limit_bytes=48 * 1024 * 1024),
    )(nodes_bf, w_cat)

    g0 = jnp.take(proj_a, edge_index[0], axis=0)
    g1 = jnp.take(proj_b, edge_index[1], axis=0)

    tile = 512
    m_pad = _round_up(m, tile)
    pad = m_pad - m
    g0 = jnp.pad(g0, ((0, pad), (0, 0)))
    g1 = jnp.pad(g1, ((0, pad), (0, 0)))
    ee_bf = jnp.pad(edges_embedding, ((0, pad), (0, 0))).astype(jnp.bfloat16)
    we_bf = w_edge.astype(jnp.bfloat16)
    bn = b_node.reshape(1, d).astype(jnp.float32)
    be = b_edge.reshape(1, d).astype(jnp.float32)

    n_hat, e_hat, absin = pl.pallas_call(
        _normalize_kernel(inv_t, m, tile, pad > 0),
        out_shape=(
            jax.ShapeDtypeStruct((m_pad, d), jnp.bfloat16),
            jax.ShapeDtypeStruct((m_pad, d), jnp.bfloat16),
            jax.ShapeDtypeStruct((m_pad, 1), jnp.float32),
        ),
        grid=(m_pad // tile,),
        in_specs=[
            pl.BlockSpec((tile, d), lambda i: (i, 0)),
            pl.BlockSpec((tile, d), lambda i: (i, 0)),
            pl.BlockSpec((tile, e_dim), lambda i: (i, 0)),
            pl.BlockSpec((e_dim, d), lambda i: (0, 0)),
            pl.BlockSpec((1, d), lambda i: (0, 0)),
            pl.BlockSpec((1, d), lambda i: (0, 0)),
        ],
        out_specs=(
            pl.BlockSpec((tile, d), lambda i: (i, 0)),
            pl.BlockSpec((tile, d), lambda i: (i, 0)),
            pl.BlockSpec((tile, 1), lambda i: (i, 0)),
        ),
        compiler_params=pltpu.CompilerParams(
            dimension_semantics=("parallel",),
            vmem_limit_bytes=48 * 1024 * 1024),
    )(g0, g1, ee_bf, we_bf, bn, be)

    tm = tn = 512
    num_i = m_pad // tm
    num_j = m_pad // tn
    rowsum, colpart = pl.pallas_call(
        _similarity_kernel(tn),
        out_shape=(
            jax.ShapeDtypeStruct((m_pad, 1), jnp.float32),
            jax.ShapeDtypeStruct((num_i, 1, m_pad), jnp.float32),
        ),
        grid=(num_i, num_j),
        in_specs=[
            pl.BlockSpec((tm, d), lambda i, j: (i, 0)),
            pl.BlockSpec((m_pad, d), lambda i, j: (0, 0)),
        ],
        out_specs=(
            pl.BlockSpec((tm, 1), lambda i, j: (i, 0)),
            pl.BlockSpec((1, 1, tn), lambda i, j: (i, 0, j)),
        ),
        compiler_params=pltpu.CompilerParams(
            dimension_semantics=("parallel", "arbitrary"),
            vmem_limit_bytes=48 * 1024 * 1024),
    )(n_hat, e_hat)

    colsum = jnp.sum(colpart, axis=(0, 1))[:m]
    denom = rowsum[:m, 0] + colsum - 2.0 * pad
    loss = absin[:m, 0] - math.log(2.0) + jnp.log(denom)
    return loss


# speedup vs baseline: 6.7178x; 1.8140x over previous
import math

import jax
import jax.numpy as jnp
from jax import lax
from jax.experimental import pallas as pl
from jax.experimental.pallas import tpu as pltpu

_LOG2E = 1.4426950408889634
_TEMPERATURE = 0.7


def _round_up(x, m):
    return ((x + m - 1) // m) * m


def _node_project_kernel(x_ref, w_ref, a_ref, b_ref):
    ab = jnp.dot(x_ref[...].astype(jnp.bfloat16), w_ref[...],
                 preferred_element_type=jnp.float32)
    d = a_ref.shape[1]
    a_ref[...] = ab[:, :d].astype(a_ref.dtype)
    b_ref[...] = ab[:, d:].astype(b_ref.dtype)


def _normalize_kernel(inv_t, m_actual, tb, mask_rows):
    s2 = math.sqrt(_LOG2E * inv_t)

    def _body(g0_ref, g1_ref, ee_ref, we_ref, bn_ref, be_ref,
              nhat_ref, ehat_ref, absin_ref):
        nodes_map = (g0_ref[...].astype(jnp.float32)
                     + g1_ref[...].astype(jnp.float32) + bn_ref[...])
        edges_map = jnp.dot(ee_ref[...].astype(jnp.bfloat16), we_ref[...],
                            preferred_element_type=jnp.float32) + be_ref[...]

        n_sq = jnp.sum(nodes_map * nodes_map, axis=-1, keepdims=True)
        e_sq = jnp.sum(edges_map * edges_map, axis=-1, keepdims=True)
        n_scale = jnp.where(n_sq > 0.0, lax.rsqrt(n_sq), 0.0)
        e_scale = jnp.where(e_sq > 0.0, lax.rsqrt(e_sq), 0.0)

        if mask_rows:
            row0 = pl.program_id(0) * tb
            valid = (row0 + lax.broadcasted_iota(jnp.int32, (tb, 1), 0)
                     < m_actual).astype(jnp.float32)
            n_scale = n_scale * valid
            e_scale = e_scale * valid

        rowdot = jnp.sum(nodes_map * edges_map, axis=-1, keepdims=True)
        absin_ref[...] = jnp.abs(rowdot) * (n_scale * e_scale * inv_t)
        nhat_ref[...] = (nodes_map * (n_scale * s2)).astype(nhat_ref.dtype)
        ehat_ref[...] = (edges_map * (e_scale * s2)).astype(ehat_ref.dtype)

    return _body


def _similarity_kernel(tn, num_j):
    def _body(nhat_ref, ehat_ref, rowsum_ref, colpart_ref):
        n_tile = nhat_ref[...]
        acc = jnp.zeros((n_tile.shape[0], 1), jnp.float32)
        for jc in range(num_j):
            e_chunk = ehat_ref[jc * tn:(jc + 1) * tn, :]
            mat = lax.dot_general(
                n_tile, e_chunk,
                dimension_numbers=(((1,), (1,)), ((), ())),
                preferred_element_type=jnp.float32)
            mi = jnp.exp2(-jnp.abs(mat))
            acc = acc + jnp.sum(mi, axis=1, keepdims=True)
            colpart_ref[0, 0, jc * tn:(jc + 1) * tn] = jnp.sum(mi, axis=0)
        rowsum_ref[...] = acc

    return _body


def kernel(nodes_embedding, edges_embedding, edge_index,
           w_node, b_node, w_edge, b_edge):
    n_nodes, e_dim = nodes_embedding.shape
    m = edge_index.shape[1]
    d = w_node.shape[1]
    inv_t = float(1.0 / _TEMPERATURE)

    tb_n = 256
    n_pad = _round_up(n_nodes, tb_n)
    nodes_p = jnp.pad(nodes_embedding, ((0, n_pad - n_nodes), (0, 0)))
    w_cat = jnp.concatenate([w_node[:e_dim], w_node[e_dim:]],
                            axis=1).astype(jnp.bfloat16)
    proj_a, proj_b = pl.pallas_call(
        _node_project_kernel,
        out_shape=(
            jax.ShapeDtypeStruct((n_pad, d), jnp.bfloat16),
            jax.ShapeDtypeStruct((n_pad, d), jnp.bfloat16),
        ),
        grid=(n_pad // tb_n,),
        in_specs=[
            pl.BlockSpec((tb_n, e_dim), lambda i: (i, 0)),
            pl.BlockSpec((e_dim, 2 * d), lambda i: (0, 0)),
        ],
        out_specs=(
            pl.BlockSpec((tb_n, d), lambda i: (i, 0)),
            pl.BlockSpec((tb_n, d), lambda i: (i, 0)),
        ),
        compiler_params=pltpu.CompilerParams(
            dimension_semantics=("parallel",),
            vmem_limit_bytes=48 * 1024 * 1024),
    )(nodes_p, w_cat)

    g0 = jnp.take(proj_a, edge_index[0], axis=0)
    g1 = jnp.take(proj_b, edge_index[1], axis=0)

    tile = 512
    m_pad = _round_up(m, tile)
    pad = m_pad - m
    g0 = jnp.pad(g0, ((0, pad), (0, 0)))
    g1 = jnp.pad(g1, ((0, pad), (0, 0)))
    ee_p = jnp.pad(edges_embedding, ((0, pad), (0, 0)))
    we_bf = w_edge.astype(jnp.bfloat16)
    bn = b_node.reshape(1, d).astype(jnp.float32)
    be = b_edge.reshape(1, d).astype(jnp.float32)

    n_hat, e_hat, absin = pl.pallas_call(
        _normalize_kernel(inv_t, m, tile, pad > 0),
        out_shape=(
            jax.ShapeDtypeStruct((m_pad, d), jnp.bfloat16),
            jax.ShapeDtypeStruct((m_pad, d), jnp.bfloat16),
            jax.ShapeDtypeStruct((m_pad, 1), jnp.float32),
        ),
        grid=(m_pad // tile,),
        in_specs=[
            pl.BlockSpec((tile, d), lambda i: (i, 0)),
            pl.BlockSpec((tile, d), lambda i: (i, 0)),
            pl.BlockSpec((tile, e_dim), lambda i: (i, 0)),
            pl.BlockSpec((e_dim, d), lambda i: (0, 0)),
            pl.BlockSpec((1, d), lambda i: (0, 0)),
            pl.BlockSpec((1, d), lambda i: (0, 0)),
        ],
        out_specs=(
            pl.BlockSpec((tile, d), lambda i: (i, 0)),
            pl.BlockSpec((tile, d), lambda i: (i, 0)),
            pl.BlockSpec((tile, 1), lambda i: (i, 0)),
        ),
        compiler_params=pltpu.CompilerParams(
            dimension_semantics=("parallel",),
            vmem_limit_bytes=48 * 1024 * 1024),
    )(g0, g1, ee_p, we_bf, bn, be)

    tm = tn = 512
    num_i = m_pad // tm
    num_j = m_pad // tn
    rowsum, colpart = pl.pallas_call(
        _similarity_kernel(tn, num_j),
        out_shape=(
            jax.ShapeDtypeStruct((m_pad, 1), jnp.float32),
            jax.ShapeDtypeStruct((num_i, 1, m_pad), jnp.float32),
        ),
        grid=(num_i,),
        in_specs=[
            pl.BlockSpec((tm, d), lambda i: (i, 0)),
            pl.BlockSpec((m_pad, d), lambda i: (0, 0)),
        ],
        out_specs=(
            pl.BlockSpec((tm, 1), lambda i: (i, 0)),
            pl.BlockSpec((1, 1, m_pad), lambda i: (i, 0, 0)),
        ),
        compiler_params=pltpu.CompilerParams(
            dimension_semantics=("parallel",),
            vmem_limit_bytes=48 * 1024 * 1024),
    )(n_hat, e_hat)

    colsum = jnp.sum(colpart, axis=(0, 1))[:m]
    denom = rowsum[:m, 0] + colsum - 2.0 * pad
    loss = absin[:m, 0] - math.log(2.0) + jnp.log(denom)
    return loss


# speedup vs baseline: 9.4018x; 1.3995x over previous
import math

import jax
import jax.numpy as jnp
from jax import lax
from jax.experimental import pallas as pl
from jax.experimental.pallas import tpu as pltpu

_LOG2E = 1.4426950408889634
_TEMPERATURE = 0.7


def _round_up(x, m):
    return ((x + m - 1) // m) * m


def _ehat_kernel(inv_t, m_actual, tb, mask_rows):
    s2 = math.sqrt(_LOG2E * inv_t)

    def _body(ee_ref, we_ref, be_ref, ehat_ref):
        edges_map = jnp.dot(ee_ref[...].astype(jnp.bfloat16), we_ref[...],
                            preferred_element_type=jnp.float32) + be_ref[...]
        e_sq = jnp.sum(edges_map * edges_map, axis=-1, keepdims=True)
        e_scale = jnp.where(e_sq > 0.0, lax.rsqrt(e_sq), 0.0)
        if mask_rows:
            row0 = pl.program_id(0) * tb
            valid = (row0 + lax.broadcasted_iota(jnp.int32, (tb, 1), 0)
                     < m_actual).astype(jnp.float32)
            e_scale = e_scale * valid
        ehat_ref[...] = (edges_map * (e_scale * s2)).astype(ehat_ref.dtype)

    return _body


def _similarity_kernel(inv_t, m_actual, tm, tn, num_j, n_pad, mask_rows):
    s2 = math.sqrt(_LOG2E * inv_t)

    def _gather_row(table_ref, base, shift):
        chunk = table_ref[pl.ds(pl.multiple_of(base, 8), 8), :]
        return pltpu.roll(chunk, shift, axis=0)[0:1, :]

    def _body(base0_ref, shift0_ref, base1_ref, shift1_ref,
              nodes_ref, wcat_ref, bn_ref, ehat_ref,
              rowsum_ref, colsum_ref, absin_ref,
              rp_scratch, nm_a, nm_b, pa_ref, pb_ref):
        i = pl.program_id(0)
        d = pa_ref.shape[1]

        @pl.when(i == 0)
        def _():
            colsum_ref[...] = jnp.zeros_like(colsum_ref)
            for t in range(n_pad // 256):
                sl = slice(t * 256, (t + 1) * 256)
                ab = jnp.dot(nodes_ref[sl, :].astype(jnp.bfloat16),
                             wcat_ref[...],
                             preferred_element_type=jnp.float32)
                pa_ref[sl, :] = ab[:, :d]
                pb_ref[sl, :] = ab[:, d:]

        half = tm // 2

        def _gather_tile(row_base):
            for h, nm_scratch in ((0, nm_a), (1, nm_b)):
                off = row_base + h * half
                for mi_ in range(half):
                    g0 = _gather_row(pa_ref, base0_ref[off + mi_],
                                     shift0_ref[off + mi_])
                    g1 = _gather_row(pb_ref, base1_ref[off + mi_],
                                     shift1_ref[off + mi_])
                    nm_scratch[mi_:mi_ + 1, :] = g0 + g1

        @pl.when(i == 0)
        def _():
            _gather_tile(0)

        row0 = i * tm
        nodes_map = (jnp.concatenate([nm_a[...], nm_b[...]], axis=0)
                     + bn_ref[...])
        n_sq = jnp.sum(nodes_map * nodes_map, axis=-1, keepdims=True)
        n_scale = jnp.where(n_sq > 0.0, lax.rsqrt(n_sq), 0.0)
        if mask_rows:
            valid = (row0 + lax.broadcasted_iota(jnp.int32, (tm, 1), 0)
                     < m_actual).astype(jnp.float32)
            n_scale = n_scale * valid
        n2f = nodes_map * (n_scale * s2)
        n_tile = n2f.astype(ehat_ref.dtype)

        e_diag = ehat_ref[pl.ds(pl.multiple_of(row0, tm), tm), :]
        rowdot = jnp.sum(n2f * e_diag.astype(jnp.float32), axis=-1,
                         keepdims=True)
        absin_ref[...] = jnp.abs(rowdot) * (1.0 / _LOG2E)

        next_row0 = jnp.where(i + 1 < pl.num_programs(0), row0 + tm, 0)
        _gather_tile(next_row0)

        for jc in range(num_j):
            e_chunk = ehat_ref[jc * tn:(jc + 1) * tn, :]
            mat = lax.dot_general(
                n_tile, e_chunk,
                dimension_numbers=(((1,), (1,)), ((), ())),
                preferred_element_type=jnp.float32)
            mi = jnp.exp2(-jnp.abs(mat))
            rp_scratch[:, jc:jc + 1] = jnp.sum(mi, axis=1, keepdims=True)
            colsum_ref[0, jc * tn:(jc + 1) * tn] += jnp.sum(mi, axis=0)
        rowsum_ref[...] = jnp.sum(rp_scratch[...], axis=1, keepdims=True)

    return _body


def kernel(nodes_embedding, edges_embedding, edge_index,
           w_node, b_node, w_edge, b_edge):
    n_nodes, e_dim = nodes_embedding.shape
    m = edge_index.shape[1]
    d = w_node.shape[1]
    inv_t = float(1.0 / _TEMPERATURE)

    tile = 512
    m_pad = _round_up(m, tile)
    pad = m_pad - m
    n_pad = _round_up(n_nodes, 256)
    nodes_p = jnp.pad(nodes_embedding, ((0, n_pad - n_nodes), (0, 0)))
    w_cat = jnp.concatenate([w_node[:e_dim], w_node[e_dim:]],
                            axis=1).astype(jnp.bfloat16)
    idx0 = jnp.pad(edge_index[0], (0, pad))
    idx1 = jnp.pad(edge_index[1], (0, pad))
    base0 = (idx0 >> 3) << 3
    shift0 = (-idx0) & 7
    base1 = (idx1 >> 3) << 3
    shift1 = (-idx1) & 7
    ee_p = jnp.pad(edges_embedding, ((0, pad), (0, 0)))
    we_bf = w_edge.astype(jnp.bfloat16)
    bn = b_node.reshape(1, d).astype(jnp.float32)
    be = b_edge.reshape(1, d).astype(jnp.float32)

    e_hat = pl.pallas_call(
        _ehat_kernel(inv_t, m, tile, pad > 0),
        out_shape=jax.ShapeDtypeStruct((m_pad, d), jnp.float8_e4m3fn),
        grid=(m_pad // tile,),
        in_specs=[
            pl.BlockSpec((tile, e_dim), lambda i: (i, 0)),
            pl.BlockSpec((e_dim, d), lambda i: (0, 0)),
            pl.BlockSpec((1, d), lambda i: (0, 0)),
        ],
        out_specs=pl.BlockSpec((tile, d), lambda i: (i, 0)),
        compiler_params=pltpu.CompilerParams(
            dimension_semantics=("parallel",),
            vmem_limit_bytes=48 * 1024 * 1024),
    )(ee_p, we_bf, be)

    tm = tn = tile
    num_i = m_pad // tm
    num_j = m_pad // tn
    rowsum, colsum, absin = pl.pallas_call(
        _similarity_kernel(inv_t, m, tm, tn, num_j, n_pad, pad > 0),
        grid_spec=pltpu.PrefetchScalarGridSpec(
            num_scalar_prefetch=4,
            grid=(num_i,),
            in_specs=[
                pl.BlockSpec((n_pad, e_dim), lambda i, *_: (0, 0)),
                pl.BlockSpec((e_dim, 2 * d), lambda i, *_: (0, 0)),
                pl.BlockSpec((1, d), lambda i, *_: (0, 0)),
                pl.BlockSpec((m_pad, d), lambda i, *_: (0, 0)),
            ],
            out_specs=(
                pl.BlockSpec((tm, 1), lambda i, *_: (i, 0)),
                pl.BlockSpec((1, m_pad), lambda i, *_: (0, 0)),
                pl.BlockSpec((tm, 1), lambda i, *_: (i, 0)),
            ),
            scratch_shapes=[pltpu.VMEM((tm, num_j), jnp.float32),
                            pltpu.VMEM((tm // 2, d), jnp.float32),
                            pltpu.VMEM((tm // 2, d), jnp.float32),
                            pltpu.VMEM((n_pad, d), jnp.float32),
                            pltpu.VMEM((n_pad, d), jnp.float32)],
        ),
        out_shape=(
            jax.ShapeDtypeStruct((m_pad, 1), jnp.float32),
            jax.ShapeDtypeStruct((1, m_pad), jnp.float32),
            jax.ShapeDtypeStruct((m_pad, 1), jnp.float32),
        ),
        compiler_params=pltpu.CompilerParams(
            dimension_semantics=("arbitrary",),
            vmem_limit_bytes=48 * 1024 * 1024),
    )(base0, shift0, base1, shift1, nodes_p, w_cat, bn, e_hat)

    denom = rowsum[:m, 0] + colsum[0, :m] - 2.0 * pad
    loss = absin[:m, 0] - math.log(2.0) + jnp.log(denom)
    return loss


# speedup vs baseline: 9.5206x; 1.0126x over previous
import math

import jax
import jax.numpy as jnp
from jax import lax
from jax.experimental import pallas as pl
from jax.experimental.pallas import tpu as pltpu

_LOG2E = 1.4426950408889634
_TEMPERATURE = 0.7


def _round_up(x, m):
    return ((x + m - 1) // m) * m


def _normalize_kernel(inv_t, m_actual, tb, mask_rows, n_pad):
    s2 = math.sqrt(_LOG2E * inv_t)

    def _gather_row(table_ref, base, shift):
        chunk = table_ref[pl.ds(pl.multiple_of(base, 8), 8), :]
        return pltpu.roll(chunk, shift, axis=0)[0:1, :]

    def _body(base0_ref, shift0_ref, base1_ref, shift1_ref,
              nodes_ref, wcat_ref, ee_ref, we_ref,
              bn_ref, be_ref, nhat_ref, ehat_ref, absin_ref,
              nm_a, nm_b, pa_ref, pb_ref):
        d = pa_ref.shape[1]

        @pl.when(pl.program_id(0) == 0)
        def _():
            for t in range(n_pad // 256):
                sl = slice(t * 256, (t + 1) * 256)
                ab = jnp.dot(nodes_ref[sl, :].astype(jnp.bfloat16),
                             wcat_ref[...],
                             preferred_element_type=jnp.float32)
                pa_ref[sl, :] = ab[:, :d]
                pb_ref[sl, :] = ab[:, d:]

        row0 = pl.program_id(0) * tb
        half = tb // 2
        for h, nm_scratch in ((0, nm_a), (1, nm_b)):
            off = row0 + h * half
            for mi in range(half):
                g0 = _gather_row(pa_ref, base0_ref[off + mi],
                                 shift0_ref[off + mi])
                g1 = _gather_row(pb_ref, base1_ref[off + mi],
                                 shift1_ref[off + mi])
                nm_scratch[mi:mi + 1, :] = g0 + g1

        nodes_map = (jnp.concatenate([nm_a[...], nm_b[...]], axis=0)
                     + bn_ref[...])
        edges_map = jnp.dot(ee_ref[...].astype(jnp.bfloat16), we_ref[...],
                            preferred_element_type=jnp.float32) + be_ref[...]

        n_sq = jnp.sum(nodes_map * nodes_map, axis=-1, keepdims=True)
        e_sq = jnp.sum(edges_map * edges_map, axis=-1, keepdims=True)
        n_scale = jnp.where(n_sq > 0.0, lax.rsqrt(n_sq), 0.0)
        e_scale = jnp.where(e_sq > 0.0, lax.rsqrt(e_sq), 0.0)

        if mask_rows:
            valid = (row0 + lax.broadcasted_iota(jnp.int32, (tb, 1), 0)
                     < m_actual).astype(jnp.float32)
            n_scale = n_scale * valid
            e_scale = e_scale * valid

        rowdot = jnp.sum(nodes_map * edges_map, axis=-1, keepdims=True)
        absin_ref[...] = jnp.abs(rowdot) * (n_scale * e_scale * inv_t)
        nhat_ref[...] = (nodes_map * (n_scale * s2)).astype(nhat_ref.dtype)
        ehat_ref[...] = (edges_map * (e_scale * s2)).astype(ehat_ref.dtype)

    return _body


def _similarity_kernel(tn, num_j):
    def _body(nhat_ref, ehat_ref, rowsum_ref, colsum_ref, rp_scratch):
        i = pl.program_id(0)

        @pl.when(i == 0)
        def _():
            colsum_ref[...] = jnp.zeros_like(colsum_ref)

        n_tile = nhat_ref[...]
        for jc in range(num_j):
            e_chunk = ehat_ref[jc * tn:(jc + 1) * tn, :]
            mat = lax.dot_general(
                n_tile, e_chunk,
                dimension_numbers=(((1,), (1,)), ((), ())),
                preferred_element_type=jnp.float32)
            mi = jnp.exp2(-jnp.abs(mat))
            rp_scratch[:, jc:jc + 1] = jnp.sum(mi, axis=1, keepdims=True)
            colsum_ref[0, jc * tn:(jc + 1) * tn] += jnp.sum(mi, axis=0)
        rowsum_ref[...] = jnp.sum(rp_scratch[...], axis=1, keepdims=True)

    return _body


def kernel(nodes_embedding, edges_embedding, edge_index,
           w_node, b_node, w_edge, b_edge):
    n_nodes, e_dim = nodes_embedding.shape
    m = edge_index.shape[1]
    d = w_node.shape[1]
    inv_t = float(1.0 / _TEMPERATURE)

    tb_n = 256
    n_pad = _round_up(n_nodes, tb_n)
    nodes_p = jnp.pad(nodes_embedding, ((0, n_pad - n_nodes), (0, 0)))
    w_cat = jnp.concatenate([w_node[:e_dim], w_node[e_dim:]],
                            axis=1).astype(jnp.bfloat16)
    tile = 512
    m_pad = _round_up(m, tile)
    pad = m_pad - m
    idx0 = jnp.pad(edge_index[0], (0, pad))
    idx1 = jnp.pad(edge_index[1], (0, pad))
    base0 = (idx0 >> 3) << 3
    shift0 = (-idx0) & 7
    base1 = (idx1 >> 3) << 3
    shift1 = (-idx1) & 7
    ee_p = jnp.pad(edges_embedding, ((0, pad), (0, 0)))
    we_bf = w_edge.astype(jnp.bfloat16)
    bn = b_node.reshape(1, d).astype(jnp.float32)
    be = b_edge.reshape(1, d).astype(jnp.float32)

    n_hat, e_hat, absin = pl.pallas_call(
        _normalize_kernel(inv_t, m, tile, pad > 0, n_pad),
        grid_spec=pltpu.PrefetchScalarGridSpec(
            num_scalar_prefetch=4,
            grid=(m_pad // tile,),
            in_specs=[
                pl.BlockSpec((n_pad, e_dim), lambda i, *_: (0, 0)),
                pl.BlockSpec((e_dim, 2 * d), lambda i, *_: (0, 0)),
                pl.BlockSpec((tile, e_dim), lambda i, *_: (i, 0)),
                pl.BlockSpec((e_dim, d), lambda i, *_: (0, 0)),
                pl.BlockSpec((1, d), lambda i, *_: (0, 0)),
                pl.BlockSpec((1, d), lambda i, *_: (0, 0)),
            ],
            out_specs=(
                pl.BlockSpec((tile, d), lambda i, *_: (i, 0)),
                pl.BlockSpec((tile, d), lambda i, *_: (i, 0)),
                pl.BlockSpec((tile, 1), lambda i, *_: (i, 0)),
            ),
            scratch_shapes=[pltpu.VMEM((tile // 2, d), jnp.float32),
                            pltpu.VMEM((tile // 2, d), jnp.float32),
                            pltpu.VMEM((n_pad, d), jnp.float32),
                            pltpu.VMEM((n_pad, d), jnp.float32)],
        ),
        out_shape=(
            jax.ShapeDtypeStruct((m_pad, d), jnp.float8_e4m3fn),
            jax.ShapeDtypeStruct((m_pad, d), jnp.float8_e4m3fn),
            jax.ShapeDtypeStruct((m_pad, 1), jnp.float32),
        ),
        compiler_params=pltpu.CompilerParams(
            dimension_semantics=("arbitrary",),
            vmem_limit_bytes=48 * 1024 * 1024),
    )(base0, shift0, base1, shift1, nodes_p, w_cat, ee_p, we_bf, bn, be)

    tm = tn = 512
    num_i = m_pad // tm
    num_j = m_pad // tn
    rowsum, colsum = pl.pallas_call(
        _similarity_kernel(tn, num_j),
        out_shape=(
            jax.ShapeDtypeStruct((m_pad, 1), jnp.float32),
            jax.ShapeDtypeStruct((1, m_pad), jnp.float32),
        ),
        grid=(num_i,),
        in_specs=[
            pl.BlockSpec((tm, d), lambda i: (i, 0)),
            pl.BlockSpec((m_pad, d), lambda i: (0, 0)),
        ],
        out_specs=(
            pl.BlockSpec((tm, 1), lambda i: (i, 0)),
            pl.BlockSpec((1, m_pad), lambda i: (0, 0)),
        ),
        scratch_shapes=[pltpu.VMEM((tm, num_j), jnp.float32)],
        compiler_params=pltpu.CompilerParams(
            dimension_semantics=("arbitrary",),
            vmem_limit_bytes=48 * 1024 * 1024),
    )(n_hat, e_hat)

    denom = rowsum[:m, 0] + colsum[0, :m] - 2.0 * pad
    loss = absin[:m, 0] - math.log(2.0) + jnp.log(denom)
    return loss
